# baseline (speedup 1.0000x reference)
import functools

import jax
import jax.numpy as jnp
import numpy as np
from jax.experimental import pallas as pl

B, T, C = 1, 2048, 2048
H, HKV, HD = 16, 4, 128
HI, DI = 16, 32
LOCAL = 128
TOPK = max(int(T * 0.5), LOCAL)
RF = H // HKV

NEG = np.float32(-1e9)
POS = np.float32(1e9)
LOG_EPS = np.log(np.float32(1e-6)).astype(np.float32)
EPS32 = np.float32(np.finfo(np.float32).eps)
INT_MIN = np.int32(-2147483648)
INT_MAX = np.int32(2147483647)

BQ = 1024


def _mm_kernel(x_ref, w_ref, o_ref, *, precision):
    o_ref[...] = jnp.dot(x_ref[...], w_ref[...],
                         preferred_element_type=jnp.float32,
                         precision=precision)


def _mm_bf16_kernel(x_ref, w_ref, o_ref):
    o_ref[...] = jnp.dot(x_ref[...].astype(jnp.bfloat16),
                         w_ref[...].astype(jnp.bfloat16),
                         preferred_element_type=jnp.float32
                         ).astype(jnp.bfloat16)


def _matmul_bf16(x, w, bm, bn):
    M, K = x.shape
    _, N = w.shape
    return pl.pallas_call(
        _mm_bf16_kernel,
        grid=(M // bm, N // bn),
        in_specs=[pl.BlockSpec((bm, K), lambda i, j: (i, 0)),
                  pl.BlockSpec((K, bn), lambda i, j: (0, j))],
        out_specs=pl.BlockSpec((bm, bn), lambda i, j: (i, j)),
        out_shape=jax.ShapeDtypeStruct((M, N), jnp.bfloat16),
    )(x, w)


def _matmul(x, w, bm, bn, precision=None):
    M, K = x.shape
    _, N = w.shape
    return pl.pallas_call(
        functools.partial(_mm_kernel, precision=precision),
        grid=(M // bm, N // bn),
        in_specs=[pl.BlockSpec((bm, K), lambda i, j: (i, 0)),
                  pl.BlockSpec((K, bn), lambda i, j: (0, j))],
        out_specs=pl.BlockSpec((bm, bn), lambda i, j: (i, j)),
        out_shape=jax.ShapeDtypeStruct((M, N), jnp.float32),
    )(x, w)


def _rot_rms_kernel(x_ref, cos_ref, sin_ref, o_ref):
    xb = x_ref[...]
    x1 = xb[:, :HD // 2]
    x2 = xb[:, HD // 2:]
    cos = cos_ref[...]
    sin = sin_ref[...]
    y1 = x1 * cos + x2 * sin
    y2 = x2 * cos - x1 * sin
    y = jnp.concatenate([y1, y2], axis=1)
    ms = jnp.mean(y * y, axis=1, keepdims=True)
    o_ref[...] = y * jax.lax.rsqrt(ms + EPS32)


def _rot_rms(x, cos, sin, nheads):
    return pl.pallas_call(
        _rot_rms_kernel,
        grid=(T // BQ, nheads),
        in_specs=[pl.BlockSpec((BQ, HD), lambda i, h: (i, h)),
                  pl.BlockSpec((BQ, HD // 2), lambda i, h: (i, 0)),
                  pl.BlockSpec((BQ, HD // 2), lambda i, h: (i, 0))],
        out_specs=pl.BlockSpec((BQ, HD), lambda i, h: (i, h)),
        out_shape=jax.ShapeDtypeStruct((T, nheads * HD), jnp.float32),
    )(x, cos, sin)


def _select_kernel(imp_ref, bias_ref):
    qb = pl.program_id(0)
    imp = imp_ref[...]
    rows = qb * BQ + jax.lax.broadcasted_iota(jnp.int32, (BQ, T), 0)
    cols = jax.lax.broadcasted_iota(jnp.int32, (BQ, T), 1)
    above = cols > rows
    dist = cols - rows
    local = (dist >= 0) & (dist < LOCAL)
    imp = jnp.where(above, NEG, imp)
    imp = jnp.where(local, POS, imp)

    u = jax.lax.bitcast_convert_type(imp + jnp.float32(0.0), jnp.int32)
    key = jnp.where(u < 0, u ^ INT_MAX, u)

    def vbody(_, carry):
        lo, hi = carry
        mid = (lo >> 1) + (hi >> 1) + (lo & hi & 1)
        cnt = jnp.sum((key >= mid).astype(jnp.int32), axis=1, keepdims=True)
        ge = cnt >= TOPK
        return jnp.where(ge, mid, lo), jnp.where(ge, hi, mid)

    lo = jnp.full((BQ, 1), INT_MIN, jnp.int32)
    hi = jnp.full((BQ, 1), INT_MAX, jnp.int32)
    lo, hi = jax.lax.fori_loop(0, 33, vbody, (lo, hi))
    t = lo

    gt = key > t
    eq = key == t
    cgt = jnp.sum(gt.astype(jnp.int32), axis=1, keepdims=True)
    need = TOPK - cgt

    def ibody(_, carry):
        lo_i, hi_i = carry
        mid = (lo_i + hi_i) >> 1
        cnt = jnp.sum((eq & (cols <= mid)).astype(jnp.int32),
                      axis=1, keepdims=True)
        ge = cnt >= need
        return jnp.where(ge, lo_i, mid), jnp.where(ge, mid, hi_i)

    lo_i = jnp.full((BQ, 1), -1, jnp.int32)
    hi_i = jnp.full((BQ, 1), T - 1, jnp.int32)
    lo_i, hi_i = jax.lax.fori_loop(0, 12, ibody, (lo_i, hi_i))
    cutoff = hi_i

    sel = gt | (eq & (cols <= cutoff))
    hard = (sel & (cols <= rows)) | (cols == rows)
    bias_ref[...] = jnp.where(hard, jnp.float32(0.0), LOG_EPS)


def _select_bias(imp):
    return pl.pallas_call(
        _select_kernel,
        grid=(T // BQ,),
        in_specs=[pl.BlockSpec((BQ, T), lambda i: (i, 0))],
        out_specs=pl.BlockSpec((BQ, T), lambda i: (i, 0)),
        out_shape=jax.ShapeDtypeStruct((T, T), jnp.float32),
    )(imp)


def _attn_kernel(q_ref, k_ref, v_ref, b_ref, o_ref):
    q = q_ref[...]
    k = k_ref[...]
    v = v_ref[...]
    bias = b_ref[...]
    logits = jax.lax.dot_general(q, k, (((1,), (1,)), ((), ())),
                                 preferred_element_type=jnp.float32)
    logits = logits * jnp.float32(1.0 / np.sqrt(128.0)) + bias
    m = jnp.max(logits, axis=1, keepdims=True)
    p = jnp.exp(logits - m)
    s = jnp.sum(p, axis=1, keepdims=True)
    o_ref[...] = jnp.dot(p, v, preferred_element_type=jnp.float32) / s


BQA = 1024


def _attention(q, k, v, bias):
    return pl.pallas_call(
        _attn_kernel,
        grid=(T // BQA, H),
        in_specs=[pl.BlockSpec((BQA, HD), lambda i, h: (i, h)),
                  pl.BlockSpec((T, HD), lambda i, h: (0, h // RF)),
                  pl.BlockSpec((T, HD), lambda i, h: (0, h // RF)),
                  pl.BlockSpec((BQA, T), lambda i, h: (i, 0))],
        out_specs=pl.BlockSpec((BQA, HD), lambda i, h: (i, h)),
        out_shape=jax.ShapeDtypeStruct((T, H * HD), jnp.float32),
    )(q, k, v, bias)


def kernel(x, cos, sin, Wq, Wk, Wv, Wo, Wiq, Wik, Wiw):
    xf = x[0]
    cos2 = cos[0, :, 0, :]
    sin2 = sin[0, :, 0, :]

    wall = jnp.concatenate([Wq, Wk, Wv], axis=0).T
    proj = _matmul(xf, wall, 512, 512)
    qraw = proj[:, :2048]
    kraw = proj[:, 2048:2560]
    vproj = proj[:, 2560:3072]

    q = _rot_rms(qraw, cos2, sin2, H)
    k = _rot_rms(kraw, cos2, sin2, HKV)

    qib = (xf @ Wiq.T).astype(jnp.bfloat16).reshape(T, HI, DI)
    kib = (xf @ Wik.T).astype(jnp.bfloat16)
    wf = xf @ Wiw.T
    sc = jnp.einsum('qhd,kd->qhk', qib, kib,
                    preferred_element_type=jnp.float32)
    sb = jax.nn.relu(sc).astype(jnp.bfloat16)
    imp = jax.lax.dot_general(wf[:, None, :], sb,
                              (((2,), (1,)), ((0,), (0,))),
                              preferred_element_type=jnp.float32)[:, 0, :]

    bias = _select_bias(imp)

    y = _attention(q, k, vproj, bias)
    out = _matmul(y, Wo.T, 512, 512)
    return out[None]


# speedup vs baseline: 1.0149x; 1.0149x over previous
import functools

import jax
import jax.numpy as jnp
import numpy as np
from jax.experimental import pallas as pl

B, T, C = 1, 2048, 2048
H, HKV, HD = 16, 4, 128
HI, DI = 16, 32
LOCAL = 128
TOPK = max(int(T * 0.5), LOCAL)
RF = H // HKV

NEG = np.float32(-1e9)
POS = np.float32(1e9)
LOG_EPS = np.log(np.float32(1e-6)).astype(np.float32)
EPS32 = np.float32(np.finfo(np.float32).eps)
INT_MIN = np.int32(-2147483648)
INT_MAX = np.int32(2147483647)

BQ = 1024


def _mm_kernel(x_ref, w_ref, o_ref, *, precision):
    o_ref[...] = jnp.dot(x_ref[...], w_ref[...],
                         preferred_element_type=jnp.float32,
                         precision=precision)


def _mm_bf16_kernel(x_ref, w_ref, o_ref):
    o_ref[...] = jnp.dot(x_ref[...].astype(jnp.bfloat16),
                         w_ref[...].astype(jnp.bfloat16),
                         preferred_element_type=jnp.float32
                         ).astype(jnp.bfloat16)


def _matmul_bf16(x, w, bm, bn):
    M, K = x.shape
    _, N = w.shape
    return pl.pallas_call(
        _mm_bf16_kernel,
        grid=(M // bm, N // bn),
        in_specs=[pl.BlockSpec((bm, K), lambda i, j: (i, 0)),
                  pl.BlockSpec((K, bn), lambda i, j: (0, j))],
        out_specs=pl.BlockSpec((bm, bn), lambda i, j: (i, j)),
        out_shape=jax.ShapeDtypeStruct((M, N), jnp.bfloat16),
    )(x, w)


def _matmul(x, w, bm, bn, precision=None):
    M, K = x.shape
    _, N = w.shape
    return pl.pallas_call(
        functools.partial(_mm_kernel, precision=precision),
        grid=(M // bm, N // bn),
        in_specs=[pl.BlockSpec((bm, K), lambda i, j: (i, 0)),
                  pl.BlockSpec((K, bn), lambda i, j: (0, j))],
        out_specs=pl.BlockSpec((bm, bn), lambda i, j: (i, j)),
        out_shape=jax.ShapeDtypeStruct((M, N), jnp.float32),
    )(x, w)


def _rot_rms_kernel(x_ref, cos_ref, sin_ref, o_ref):
    xb = x_ref[...]
    x1 = xb[:, :HD // 2]
    x2 = xb[:, HD // 2:]
    cos = cos_ref[...]
    sin = sin_ref[...]
    y1 = x1 * cos + x2 * sin
    y2 = x2 * cos - x1 * sin
    y = jnp.concatenate([y1, y2], axis=1)
    ms = jnp.mean(y * y, axis=1, keepdims=True)
    o_ref[...] = y * jax.lax.rsqrt(ms + EPS32)


def _rot_rms(x, cos, sin, nheads):
    return pl.pallas_call(
        _rot_rms_kernel,
        grid=(T // BQ, nheads),
        in_specs=[pl.BlockSpec((BQ, HD), lambda i, h: (i, h)),
                  pl.BlockSpec((BQ, HD // 2), lambda i, h: (i, 0)),
                  pl.BlockSpec((BQ, HD // 2), lambda i, h: (i, 0))],
        out_specs=pl.BlockSpec((BQ, HD), lambda i, h: (i, h)),
        out_shape=jax.ShapeDtypeStruct((T, nheads * HD), jnp.float32),
    )(x, cos, sin)


def _select_kernel(imp_ref, bias_ref):
    qb = pl.program_id(0)
    imp = imp_ref[...]
    rows = qb * BQ + jax.lax.broadcasted_iota(jnp.int32, (BQ, T), 0)
    cols = jax.lax.broadcasted_iota(jnp.int32, (BQ, T), 1)
    above = cols > rows
    dist = cols - rows
    local = (dist >= 0) & (dist < LOCAL)
    imp = jnp.where(above, NEG, imp)
    imp = jnp.where(local, POS, imp)

    u = jax.lax.bitcast_convert_type(imp + jnp.float32(0.0), jnp.int32)
    key = jnp.where(u < 0, u ^ INT_MAX, u)

    def vbody(_, carry):
        lo, hi = carry
        mid = (lo >> 1) + (hi >> 1) + (lo & hi & 1)
        cnt = jnp.sum((key >= mid).astype(jnp.int32), axis=1, keepdims=True)
        ge = cnt >= TOPK
        return jnp.where(ge, mid, lo), jnp.where(ge, hi, mid)

    lo = jnp.full((BQ, 1), INT_MIN, jnp.int32)
    hi = jnp.full((BQ, 1), INT_MAX, jnp.int32)
    lo, hi = jax.lax.fori_loop(0, 33, vbody, (lo, hi))
    t = lo

    gt = key > t
    eq = key == t
    cgt = jnp.sum(gt.astype(jnp.int32), axis=1, keepdims=True)
    need = TOPK - cgt

    def ibody(_, carry):
        lo_i, hi_i = carry
        mid = (lo_i + hi_i) >> 1
        cnt = jnp.sum((eq & (cols <= mid)).astype(jnp.int32),
                      axis=1, keepdims=True)
        ge = cnt >= need
        return jnp.where(ge, lo_i, mid), jnp.where(ge, mid, hi_i)

    lo_i = jnp.full((BQ, 1), -1, jnp.int32)
    hi_i = jnp.full((BQ, 1), T - 1, jnp.int32)
    lo_i, hi_i = jax.lax.fori_loop(0, 12, ibody, (lo_i, hi_i))
    cutoff = hi_i

    sel = gt | (eq & (cols <= cutoff))
    hard = (sel & (cols <= rows)) | (cols == rows)
    bias_ref[...] = jnp.where(hard, jnp.float32(0.0),
                              LOG_EPS).astype(jnp.bfloat16)


def _select_bias(imp):
    return pl.pallas_call(
        _select_kernel,
        grid=(T // BQ,),
        in_specs=[pl.BlockSpec((BQ, T), lambda i: (i, 0))],
        out_specs=pl.BlockSpec((BQ, T), lambda i: (i, 0)),
        out_shape=jax.ShapeDtypeStruct((T, T), jnp.bfloat16),
    )(imp)


def _attn_kernel(q_ref, k_ref, v_ref, b_ref, o_ref):
    q = q_ref[...]
    k = k_ref[...]
    v = v_ref[...]
    bias = b_ref[...].astype(jnp.float32)
    logits = jax.lax.dot_general(q, k, (((1,), (1,)), ((), ())),
                                 preferred_element_type=jnp.float32)
    logits = logits * jnp.float32(1.0 / np.sqrt(128.0)) + bias
    m = jnp.max(logits, axis=1, keepdims=True)
    p = jnp.exp(logits - m)
    s = jnp.sum(p, axis=1, keepdims=True)
    o_ref[...] = jnp.dot(p, v, preferred_element_type=jnp.float32) / s


BQA = 1024


def _attention(q, k, v, bias):
    return pl.pallas_call(
        _attn_kernel,
        grid=(T // BQA, H),
        in_specs=[pl.BlockSpec((BQA, HD), lambda i, h: (i, h)),
                  pl.BlockSpec((T, HD), lambda i, h: (0, h // RF)),
                  pl.BlockSpec((T, HD), lambda i, h: (0, h // RF)),
                  pl.BlockSpec((BQA, T), lambda i, h: (i, 0))],
        out_specs=pl.BlockSpec((BQA, HD), lambda i, h: (i, h)),
        out_shape=jax.ShapeDtypeStruct((T, H * HD), jnp.float32),
    )(q, k, v, bias)


def kernel(x, cos, sin, Wq, Wk, Wv, Wo, Wiq, Wik, Wiw):
    xf = x[0]
    cos2 = cos[0, :, 0, :]
    sin2 = sin[0, :, 0, :]

    wall = jnp.concatenate([Wq, Wk, Wv], axis=0).T
    proj = _matmul(xf, wall, 512, 512)
    qraw = proj[:, :2048]
    kraw = proj[:, 2048:2560]
    vproj = proj[:, 2560:3072]

    q = _rot_rms(qraw, cos2, sin2, H)
    k = _rot_rms(kraw, cos2, sin2, HKV)

    qib = (xf @ Wiq.T).astype(jnp.bfloat16).reshape(T, HI, DI)
    kib = (xf @ Wik.T).astype(jnp.bfloat16)
    wf = xf @ Wiw.T
    sc = jnp.einsum('qhd,kd->qhk', qib, kib,
                    preferred_element_type=jnp.float32)
    sb = jax.nn.relu(sc).astype(jnp.bfloat16)
    imp = jax.lax.dot_general(wf[:, None, :], sb,
                              (((2,), (1,)), ((0,), (0,))),
                              preferred_element_type=jnp.float32)[:, 0, :]

    bias = _select_bias(imp)

    y = _attention(q, k, vproj, bias)
    out = _matmul(y, Wo.T, 512, 512)
    return out[None]


# speedup vs baseline: 1.0246x; 1.0095x over previous
import functools

import jax
import jax.numpy as jnp
import numpy as np
from jax.experimental import pallas as pl

B, T, C = 1, 2048, 2048
H, HKV, HD = 16, 4, 128
HI, DI = 16, 32
LOCAL = 128
TOPK = max(int(T * 0.5), LOCAL)
RF = H // HKV

NEG = np.float32(-1e9)
POS = np.float32(1e9)
LOG_EPS = np.log(np.float32(1e-6)).astype(np.float32)
EPS32 = np.float32(np.finfo(np.float32).eps)
INT_MIN = np.int32(-2147483648)
INT_MAX = np.int32(2147483647)

BQ = 1024


def _mm_kernel(x_ref, w_ref, o_ref, *, precision):
    o_ref[...] = jnp.dot(x_ref[...], w_ref[...],
                         preferred_element_type=jnp.float32,
                         precision=precision)


def _mm_bf16_kernel(x_ref, w_ref, o_ref):
    o_ref[...] = jnp.dot(x_ref[...].astype(jnp.bfloat16),
                         w_ref[...].astype(jnp.bfloat16),
                         preferred_element_type=jnp.float32
                         ).astype(jnp.bfloat16)


def _matmul_bf16(x, w, bm, bn):
    M, K = x.shape
    _, N = w.shape
    return pl.pallas_call(
        _mm_bf16_kernel,
        grid=(M // bm, N // bn),
        in_specs=[pl.BlockSpec((bm, K), lambda i, j: (i, 0)),
                  pl.BlockSpec((K, bn), lambda i, j: (0, j))],
        out_specs=pl.BlockSpec((bm, bn), lambda i, j: (i, j)),
        out_shape=jax.ShapeDtypeStruct((M, N), jnp.bfloat16),
    )(x, w)


def _matmul(x, w, bm, bn, precision=None):
    M, K = x.shape
    _, N = w.shape
    return pl.pallas_call(
        functools.partial(_mm_kernel, precision=precision),
        grid=(M // bm, N // bn),
        in_specs=[pl.BlockSpec((bm, K), lambda i, j: (i, 0)),
                  pl.BlockSpec((K, bn), lambda i, j: (0, j))],
        out_specs=pl.BlockSpec((bm, bn), lambda i, j: (i, j)),
        out_shape=jax.ShapeDtypeStruct((M, N), jnp.float32),
    )(x, w)


def _rot_rms_kernel(x_ref, cos_ref, sin_ref, o_ref):
    xb = x_ref[...]
    x1 = xb[:, :HD // 2]
    x2 = xb[:, HD // 2:]
    cos = cos_ref[...]
    sin = sin_ref[...]
    y1 = x1 * cos + x2 * sin
    y2 = x2 * cos - x1 * sin
    y = jnp.concatenate([y1, y2], axis=1)
    ms = jnp.mean(y * y, axis=1, keepdims=True)
    o_ref[...] = (y * jax.lax.rsqrt(ms + EPS32)).astype(jnp.bfloat16)


def _rot_rms(x, cos, sin, nheads):
    return pl.pallas_call(
        _rot_rms_kernel,
        grid=(T // BQ, nheads),
        in_specs=[pl.BlockSpec((BQ, HD), lambda i, h: (i, h)),
                  pl.BlockSpec((BQ, HD // 2), lambda i, h: (i, 0)),
                  pl.BlockSpec((BQ, HD // 2), lambda i, h: (i, 0))],
        out_specs=pl.BlockSpec((BQ, HD), lambda i, h: (i, h)),
        out_shape=jax.ShapeDtypeStruct((T, nheads * HD), jnp.bfloat16),
    )(x, cos, sin)


def _select_kernel(imp_ref, bias_ref):
    qb = pl.program_id(0)
    imp = imp_ref[...]
    rows = qb * BQ + jax.lax.broadcasted_iota(jnp.int32, (BQ, T), 0)
    cols = jax.lax.broadcasted_iota(jnp.int32, (BQ, T), 1)
    above = cols > rows
    dist = cols - rows
    local = (dist >= 0) & (dist < LOCAL)
    imp = jnp.where(above, NEG, imp)
    imp = jnp.where(local, POS, imp)

    u = jax.lax.bitcast_convert_type(imp + jnp.float32(0.0), jnp.int32)
    key = jnp.where(u < 0, u ^ INT_MAX, u)

    def vbody(_, carry):
        lo, hi = carry
        mid = (lo >> 1) + (hi >> 1) + (lo & hi & 1)
        cnt = jnp.sum((key >= mid).astype(jnp.int32), axis=1, keepdims=True)
        ge = cnt >= TOPK
        return jnp.where(ge, mid, lo), jnp.where(ge, hi, mid)

    lo = jnp.full((BQ, 1), INT_MIN, jnp.int32)
    hi = jnp.full((BQ, 1), INT_MAX, jnp.int32)
    lo, hi = jax.lax.fori_loop(0, 33, vbody, (lo, hi))
    t = lo

    gt = key > t
    eq = key == t
    cgt = jnp.sum(gt.astype(jnp.int32), axis=1, keepdims=True)
    need = TOPK - cgt

    def ibody(_, carry):
        lo_i, hi_i = carry
        mid = (lo_i + hi_i) >> 1
        cnt = jnp.sum((eq & (cols <= mid)).astype(jnp.int32),
                      axis=1, keepdims=True)
        ge = cnt >= need
        return jnp.where(ge, lo_i, mid), jnp.where(ge, mid, hi_i)

    lo_i = jnp.full((BQ, 1), -1, jnp.int32)
    hi_i = jnp.full((BQ, 1), T - 1, jnp.int32)
    lo_i, hi_i = jax.lax.fori_loop(0, 12, ibody, (lo_i, hi_i))
    cutoff = hi_i

    sel = gt | (eq & (cols <= cutoff))
    hard = (sel & (cols <= rows)) | (cols == rows)
    bias_ref[...] = jnp.where(hard, jnp.float32(0.0),
                              LOG_EPS).astype(jnp.bfloat16)


def _select_bias(imp):
    return pl.pallas_call(
        _select_kernel,
        grid=(T // BQ,),
        in_specs=[pl.BlockSpec((BQ, T), lambda i: (i, 0))],
        out_specs=pl.BlockSpec((BQ, T), lambda i: (i, 0)),
        out_shape=jax.ShapeDtypeStruct((T, T), jnp.bfloat16),
    )(imp)


def _attn_kernel(q_ref, k_ref, v_ref, b_ref, o_ref):
    q = q_ref[...]
    k = k_ref[...]
    v = v_ref[...]
    bias = b_ref[...].astype(jnp.float32)
    logits = jax.lax.dot_general(q, k, (((1,), (1,)), ((), ())),
                                 preferred_element_type=jnp.float32)
    logits = logits * jnp.float32(1.0 / np.sqrt(128.0)) + bias
    m = jnp.max(logits, axis=1, keepdims=True)
    p = jnp.exp(logits - m)
    s = jnp.sum(p, axis=1, keepdims=True)
    o_ref[...] = (jnp.dot(p, v, preferred_element_type=jnp.float32)
                  / s).astype(jnp.bfloat16)


BQA = 1024


def _attention(q, k, v, bias):
    return pl.pallas_call(
        _attn_kernel,
        grid=(T // BQA, H),
        in_specs=[pl.BlockSpec((BQA, HD), lambda i, h: (i, h)),
                  pl.BlockSpec((T, HD), lambda i, h: (0, h // RF)),
                  pl.BlockSpec((T, HD), lambda i, h: (0, h // RF)),
                  pl.BlockSpec((BQA, T), lambda i, h: (i, 0))],
        out_specs=pl.BlockSpec((BQA, HD), lambda i, h: (i, h)),
        out_shape=jax.ShapeDtypeStruct((T, H * HD), jnp.bfloat16),
    )(q, k, v, bias)


def kernel(x, cos, sin, Wq, Wk, Wv, Wo, Wiq, Wik, Wiw):
    xf = x[0]
    cos2 = cos[0, :, 0, :]
    sin2 = sin[0, :, 0, :]

    wall = jnp.concatenate([Wq, Wk, Wv], axis=0).T
    proj = _matmul(xf, wall, 512, 512)
    qraw = proj[:, :2048]
    kraw = proj[:, 2048:2560]
    vproj = proj[:, 2560:3072].astype(jnp.bfloat16)

    q = _rot_rms(qraw, cos2, sin2, H)
    k = _rot_rms(kraw, cos2, sin2, HKV)

    qib = (xf @ Wiq.T).astype(jnp.bfloat16).reshape(T, HI, DI)
    kib = (xf @ Wik.T).astype(jnp.bfloat16)
    wf = xf @ Wiw.T
    sc = jnp.einsum('qhd,kd->qhk', qib, kib,
                    preferred_element_type=jnp.float32)
    sb = jax.nn.relu(sc).astype(jnp.bfloat16)
    imp = jax.lax.dot_general(wf[:, None, :], sb,
                              (((2,), (1,)), ((0,), (0,))),
                              preferred_element_type=jnp.float32)[:, 0, :]

    bias = _select_bias(imp)

    y = _attention(q, k, vproj, bias)
    out = _matmul(y, Wo.T, 512, 512)
    return out[None]
